# baseline (speedup 1.0000x reference)
import functools

import jax
import jax.numpy as jnp
from jax import lax
from jax.experimental import pallas as pl
from jax.experimental.pallas import tpu as pltpu
from jax.experimental.pallas import tpu_sc as plsc

_N_BUCKETS = 4
_ROWS_PER_BUCKET = 50
_N_ENTRIES = _N_BUCKETS * _ROWS_PER_BUCKET
_EMB = 128
_CHUNK = 128


def _table_kernel(inv_ref, *refs):
    out_ref = refs[-1]
    inv = inv_ref[...]
    acc = jnp.zeros((_N_ENTRIES, _EMB), dtype=jnp.float32)
    for i in range(_N_BUCKETS):
        emb = refs[5 * i][...]
        w1 = refs[5 * i + 1][...]
        b1 = refs[5 * i + 2][...]
        w2 = refs[5 * i + 3][...]
        b2 = refs[5 * i + 4][...]
        h = jnp.maximum(
            jnp.dot(emb, w1, preferred_element_type=jnp.float32) + b1, 0.0)
        o = jnp.dot(h, w2, preferred_element_type=jnp.float32) + b2
        col = lax.broadcasted_iota(jnp.int32, (_N_ENTRIES, _ROWS_PER_BUCKET), 1)
        p = (inv == col + i * _ROWS_PER_BUCKET).astype(jnp.float32)
        acc = acc + jnp.dot(p, o, preferred_element_type=jnp.float32)
    out_ref[...] = acc


def _build_table(inv, embs, w1s, b1s, w2s, b2s):
    operands = [inv.reshape(_N_ENTRIES, 1).astype(jnp.int32)]
    for i in range(_N_BUCKETS):
        operands += [embs[i], w1s[i], b1s[i].reshape(1, -1),
                     w2s[i], b2s[i].reshape(1, -1)]
    return pl.pallas_call(
        _table_kernel,
        out_shape=jax.ShapeDtypeStruct((_N_ENTRIES, _EMB), jnp.float32),
    )(*operands)


def _sc_gather(table, idx2d):
    b, l = idx2d.shape
    info = plsc.get_sparse_core_info()
    nc, ns = info.num_cores, info.num_subcores
    nw = nc * ns
    rows_per_w = b // nw
    group = 8
    n_groups = rows_per_w // group

    mesh = plsc.VectorSubcoreMesh(core_axis_name="c", subcore_axis_name="s")

    n_pairs = n_groups // 2

    @functools.partial(
        pl.kernel,
        out_type=jax.ShapeDtypeStruct((b, l, _EMB), jnp.float32),
        mesh=mesh,
        scratch_types=[
            pltpu.VMEM((rows_per_w, l), jnp.int32),
            pltpu.VMEM((group, l, _EMB), jnp.float32),
            pltpu.VMEM((group, l, _EMB), jnp.float32),
            pltpu.SemaphoreType.DMA,
            pltpu.SemaphoreType.DMA,
            pltpu.SemaphoreType.DMA,
        ],
    )
    def gather(table_hbm, idx_hbm, out_hbm, idx_v, rows_a, rows_b, semg,
               semwa, semwb):
        wid = lax.axis_index("s") * nc + lax.axis_index("c")
        base = wid * rows_per_w
        pltpu.sync_copy(idx_hbm.at[pl.ds(base, rows_per_w)], idx_v)

        def fire(buf, g):
            return [
                pltpu.async_copy(
                    table_hbm.at[idx_v.at[g * group + i]], buf.at[i], semg)
                for i in range(group)
            ]

        def out_slice(g):
            return out_hbm.at[pl.ds(base + g * group, group)]

        def body(j, carry):
            g0 = 2 * j
            g1 = g0 + 1

            @pl.when(j >= 1)
            def _():
                pltpu.make_async_copy(rows_a, out_slice(g0 - 2), semwa).wait()

            cs = fire(rows_a, g0)
            for c in cs:
                c.wait()

            @pl.when(j >= 1)
            def _():
                pltpu.make_async_copy(rows_b, out_slice(g1 - 2), semwb).wait()

            pltpu.async_copy(rows_a, out_slice(g0), semwa)

            cs = fire(rows_b, g1)
            for c in cs:
                c.wait()
            pltpu.async_copy(rows_b, out_slice(g1), semwb)
            return carry

        lax.fori_loop(0, n_pairs, body, 0)
        pltpu.make_async_copy(rows_a, out_slice(n_groups - 2), semwa).wait()
        pltpu.make_async_copy(rows_b, out_slice(n_groups - 1), semwb).wait()

    return gather(table, idx2d)


def kernel(input, inverse_indices, emb0, W1_0, b1_0, W2_0, b2_0, emb1, W1_1,
           b1_1, W2_1, b2_1, emb2, W1_2, b1_2, W2_2, b2_2, emb3, W1_3, b1_3,
           W2_3, b2_3):
    b, l = input.shape
    table = _build_table(inverse_indices,
                         (emb0, emb1, emb2, emb3),
                         (W1_0, W1_1, W1_2, W1_3),
                         (b1_0, b1_1, b1_2, b1_3),
                         (W2_0, W2_1, W2_2, W2_3),
                         (b2_0, b2_1, b2_2, b2_3))
    return _sc_gather(table, input.astype(jnp.int32))


# speedup vs baseline: 1.6744x; 1.6744x over previous
import functools

import jax
import jax.numpy as jnp
from jax import lax
from jax.experimental import pallas as pl
from jax.experimental.pallas import tpu as pltpu
from jax.experimental.pallas import tpu_sc as plsc

_N_BUCKETS = 4
_ROWS_PER_BUCKET = 50
_N_ENTRIES = _N_BUCKETS * _ROWS_PER_BUCKET
_EMB = 128
_CHUNK = 128


def _table_kernel(inv_ref, *refs):
    out_ref = refs[-1]
    inv = inv_ref[...]
    acc = jnp.zeros((_N_ENTRIES, _EMB), dtype=jnp.float32)
    for i in range(_N_BUCKETS):
        emb = refs[5 * i][...]
        w1 = refs[5 * i + 1][...]
        b1 = refs[5 * i + 2][...]
        w2 = refs[5 * i + 3][...]
        b2 = refs[5 * i + 4][...]
        h = jnp.maximum(
            jnp.dot(emb, w1, preferred_element_type=jnp.float32) + b1, 0.0)
        o = jnp.dot(h, w2, preferred_element_type=jnp.float32) + b2
        col = lax.broadcasted_iota(jnp.int32, (_N_ENTRIES, _ROWS_PER_BUCKET), 1)
        p = (inv == col + i * _ROWS_PER_BUCKET).astype(jnp.float32)
        acc = acc + jnp.dot(p, o, preferred_element_type=jnp.float32)
    out_ref[...] = acc


def _build_table(inv, embs, w1s, b1s, w2s, b2s):
    operands = [inv.reshape(_N_ENTRIES, 1).astype(jnp.int32)]
    for i in range(_N_BUCKETS):
        operands += [embs[i], w1s[i], b1s[i].reshape(1, -1),
                     w2s[i], b2s[i].reshape(1, -1)]
    return pl.pallas_call(
        _table_kernel,
        out_shape=jax.ShapeDtypeStruct((_N_ENTRIES, _EMB), jnp.float32),
    )(*operands)


def _sc_gather(table, idx2d):
    b, l = idx2d.shape
    info = plsc.get_sparse_core_info()
    nc, ns = info.num_cores, info.num_subcores
    nw = nc * ns
    rows_per_w = b // nw
    group = 8
    n_groups = rows_per_w // group

    mesh = plsc.VectorSubcoreMesh(core_axis_name="c", subcore_axis_name="s")

    n_pairs = n_groups // 2

    @functools.partial(
        pl.kernel,
        out_type=jax.ShapeDtypeStruct((b, l, _EMB), jnp.float32),
        mesh=mesh,
        scratch_types=[
            pltpu.VMEM((rows_per_w, l), jnp.int32),
            pltpu.VMEM((group, l, _EMB), jnp.float32),
            pltpu.VMEM((group, l, _EMB), jnp.float32),
            pltpu.VMEM_SHARED((_N_ENTRIES, _EMB), jnp.float32),
            pltpu.SemaphoreType.DMA,
            pltpu.SemaphoreType.DMA,
            pltpu.SemaphoreType.DMA,
        ],
    )
    def gather(table_hbm, idx_hbm, out_hbm, idx_v, rows_a, rows_b,
               table_sp, semg, semwa, semwb):
        sid = lax.axis_index("s")
        wid = sid * nc + lax.axis_index("c")
        base = wid * rows_per_w

        @pl.when(sid == 0)
        def _():
            pltpu.sync_copy(table_hbm, table_sp)

        pltpu.sync_copy(idx_hbm.at[pl.ds(base, rows_per_w)], idx_v)
        plsc.subcore_barrier()

        def fire(buf, g):
            return [
                pltpu.async_copy(
                    table_sp.at[idx_v.at[g * group + i]], buf.at[i], semg)
                for i in range(group)
            ]

        def out_slice(g):
            return out_hbm.at[pl.ds(base + g * group, group)]

        def body(j, carry):
            g0 = 2 * j
            g1 = g0 + 1

            @pl.when(j >= 1)
            def _():
                pltpu.make_async_copy(rows_a, out_slice(g0 - 2), semwa).wait()

            cs = fire(rows_a, g0)
            for c in cs:
                c.wait()

            @pl.when(j >= 1)
            def _():
                pltpu.make_async_copy(rows_b, out_slice(g1 - 2), semwb).wait()

            pltpu.async_copy(rows_a, out_slice(g0), semwa)

            cs = fire(rows_b, g1)
            for c in cs:
                c.wait()
            pltpu.async_copy(rows_b, out_slice(g1), semwb)
            return carry

        lax.fori_loop(0, n_pairs, body, 0)
        pltpu.make_async_copy(rows_a, out_slice(n_groups - 2), semwa).wait()
        pltpu.make_async_copy(rows_b, out_slice(n_groups - 1), semwb).wait()

    return gather(table, idx2d)


def kernel(input, inverse_indices, emb0, W1_0, b1_0, W2_0, b2_0, emb1, W1_1,
           b1_1, W2_1, b2_1, emb2, W1_2, b1_2, W2_2, b2_2, emb3, W1_3, b1_3,
           W2_3, b2_3):
    b, l = input.shape
    table = _build_table(inverse_indices,
                         (emb0, emb1, emb2, emb3),
                         (W1_0, W1_1, W1_2, W1_3),
                         (b1_0, b1_1, b1_2, b1_3),
                         (W2_0, W2_1, W2_2, W2_3),
                         (b2_0, b2_1, b2_2, b2_3))
    return _sc_gather(table, input.astype(jnp.int32))


# speedup vs baseline: 1.7038x; 1.0176x over previous
import functools

import jax
import jax.numpy as jnp
from jax import lax
from jax.experimental import pallas as pl
from jax.experimental.pallas import tpu as pltpu
from jax.experimental.pallas import tpu_sc as plsc

_N_BUCKETS = 4
_ROWS_PER_BUCKET = 50
_N_ENTRIES = _N_BUCKETS * _ROWS_PER_BUCKET
_EMB = 128
_CHUNK = 128


def _table_kernel(inv_ref, *refs):
    out_ref = refs[-1]
    inv = lax.broadcast_in_dim(inv_ref[...], (_N_ENTRIES, 1), (0,))
    acc = jnp.zeros((_N_ENTRIES, _EMB), dtype=jnp.float32)
    for i in range(_N_BUCKETS):
        emb = refs[5 * i][...]
        w1 = refs[5 * i + 1][...]
        b1 = refs[5 * i + 2][...]
        w2 = refs[5 * i + 3][...]
        b2 = refs[5 * i + 4][...]
        h = jnp.maximum(
            jnp.dot(emb, w1, preferred_element_type=jnp.float32) + b1, 0.0)
        o = jnp.dot(h, w2, preferred_element_type=jnp.float32) + b2
        col = lax.broadcasted_iota(jnp.int32, (_N_ENTRIES, _ROWS_PER_BUCKET), 1)
        p = (inv == col + i * _ROWS_PER_BUCKET).astype(jnp.float32)
        acc = acc + jnp.dot(p, o, preferred_element_type=jnp.float32)
    out_ref[...] = acc


def _build_table(inv, embs, w1s, b1s, w2s, b2s):
    operands = [inv]
    for i in range(_N_BUCKETS):
        operands += [embs[i], w1s[i], b1s[i], w2s[i], b2s[i]]
    return pl.pallas_call(
        _table_kernel,
        out_shape=jax.ShapeDtypeStruct((_N_ENTRIES, _EMB), jnp.float32),
    )(*operands)


def _sc_gather(table, idx2d):
    b, l = idx2d.shape
    info = plsc.get_sparse_core_info()
    nc, ns = info.num_cores, info.num_subcores
    nw = nc * ns
    rows_per_w = b // nw
    group = 8
    n_groups = rows_per_w // group

    mesh = plsc.VectorSubcoreMesh(core_axis_name="c", subcore_axis_name="s")

    n_pairs = n_groups // 2

    @functools.partial(
        pl.kernel,
        out_type=jax.ShapeDtypeStruct((b, l, _EMB), jnp.float32),
        mesh=mesh,
        compiler_params=pltpu.CompilerParams(use_tc_tiling_on_sc=True),
        scratch_types=[
            pltpu.VMEM((rows_per_w, l), jnp.int32),
            pltpu.VMEM((group, l, _EMB), jnp.float32),
            pltpu.VMEM((group, l, _EMB), jnp.float32),
            pltpu.VMEM_SHARED((_N_ENTRIES, _EMB), jnp.float32),
            pltpu.SemaphoreType.DMA,
            pltpu.SemaphoreType.DMA,
            pltpu.SemaphoreType.DMA,
        ],
    )
    def gather(table_hbm, idx_hbm, out_hbm, idx_v, rows_a, rows_b,
               table_sp, semg, semwa, semwb):
        sid = lax.axis_index("s")
        wid = sid * nc + lax.axis_index("c")
        base = wid * rows_per_w

        @pl.when(sid == 0)
        def _():
            pltpu.sync_copy(table_hbm, table_sp)

        pltpu.sync_copy(idx_hbm.at[pl.ds(base, rows_per_w)], idx_v)
        plsc.subcore_barrier()

        def fire(buf, g):
            return [
                pltpu.async_copy(
                    table_sp.at[idx_v.at[g * group + i]], buf.at[i], semg)
                for i in range(group)
            ]

        def out_slice(g):
            return out_hbm.at[pl.ds(base + g * group, group)]

        def body(j, carry):
            g0 = 2 * j
            g1 = g0 + 1

            @pl.when(j >= 1)
            def _():
                pltpu.make_async_copy(rows_a, out_slice(g0 - 2), semwa).wait()

            cs = fire(rows_a, g0)
            for c in cs:
                c.wait()

            @pl.when(j >= 1)
            def _():
                pltpu.make_async_copy(rows_b, out_slice(g1 - 2), semwb).wait()

            pltpu.async_copy(rows_a, out_slice(g0), semwa)

            cs = fire(rows_b, g1)
            for c in cs:
                c.wait()
            pltpu.async_copy(rows_b, out_slice(g1), semwb)
            return carry

        lax.fori_loop(0, n_pairs, body, 0)
        pltpu.make_async_copy(rows_a, out_slice(n_groups - 2), semwa).wait()
        pltpu.make_async_copy(rows_b, out_slice(n_groups - 1), semwb).wait()

    return gather(table, idx2d)


def kernel(input, inverse_indices, emb0, W1_0, b1_0, W2_0, b2_0, emb1, W1_1,
           b1_1, W2_1, b2_1, emb2, W1_2, b1_2, W2_2, b2_2, emb3, W1_3, b1_3,
           W2_3, b2_3):
    b, l = input.shape
    table = _build_table(inverse_indices,
                         (emb0, emb1, emb2, emb3),
                         (W1_0, W1_1, W1_2, W1_3),
                         (b1_0, b1_1, b1_2, b1_3),
                         (W2_0, W2_1, W2_2, W2_3),
                         (b2_0, b2_1, b2_2, b2_3))
    return _sc_gather(table, input.astype(jnp.int32))
